# baseline (speedup 1.0000x reference)
import functools

import jax
import jax.numpy as jnp
from jax import lax
from jax.experimental import pallas as pl
from jax.experimental.pallas import tpu as pltpu
from jax.experimental.pallas import tpu_sc as plsc

N = 10000
E = 160000
NT = 32
CH = 128
C_CONV = 40
EP = NT * C_CONV * CH
C_DOT = 80
EP2 = NT * C_DOT * CH


def _sc_mesh():
    return plsc.VectorSubcoreMesh(core_axis_name="c", subcore_axis_name="s")


_SC_PARAMS = pltpu.CompilerParams(
    use_tc_tiling_on_sc=False, needs_layout_passes=False)


def _sc_deg(ew_rows, dst_idx, zero1):
    @functools.partial(
        pl.kernel,
        out_type=jax.ShapeDtypeStruct((2, N), jnp.float32),
        mesh=_sc_mesh(),
        compiler_params=_SC_PARAMS,
        scratch_types=[
            pltpu.VMEM((C_CONV, CH), jnp.float32),
            pltpu.VMEM((C_CONV, CH), jnp.int32),
            pltpu.VMEM_SHARED((N,), jnp.float32),
        ],
    )
    def k(ew_hbm, dst_hbm, zero_hbm, out_hbm, ew_v, dst_v, acc):
        c = lax.axis_index("c")
        s = lax.axis_index("s")
        t = s * 2 + c

        @pl.when(s == 0)
        def _():
            pltpu.sync_copy(zero_hbm, acc)

        plsc.subcore_barrier()
        pltpu.sync_copy(ew_hbm.at[t], ew_v)
        pltpu.sync_copy(dst_hbm.at[t], dst_v)

        def body(j, carry):
            pltpu.sync_copy(ew_v.at[j], acc.at[dst_v.at[j]], add=True)
            return carry

        lax.fori_loop(0, C_CONV, body, 0)
        plsc.subcore_barrier()

        @pl.when(s == 0)
        def _():
            pltpu.sync_copy(acc, out_hbm.at[c])

    return k(ew_rows, dst_idx, zero1)


def _sc_conv(tbl, src_idx, dst_idx, ew_rows, zero_w, width):
    @functools.partial(
        pl.kernel,
        out_type=jax.ShapeDtypeStruct((2, N, width), jnp.float32),
        mesh=_sc_mesh(),
        compiler_params=_SC_PARAMS,
        scratch_types=[
            pltpu.VMEM((C_CONV, CH), jnp.int32),
            pltpu.VMEM((C_CONV, CH), jnp.int32),
            pltpu.VMEM((C_CONV, CH), jnp.float32),
            pltpu.VMEM((CH, width), jnp.float32),
            pltpu.VMEM((CH, width), jnp.float32),
            pltpu.VMEM_SHARED((N, width), jnp.float32),
            pltpu.SemaphoreType.DMA,
            pltpu.SemaphoreType.DMA,
        ],
    )
    def k(tbl_hbm, src_hbm, dst_hbm, ew_hbm, zero_hbm, out_hbm,
          src_v, dst_v, ew_v, gb0, gb1, acc, sg0, sg1):
        c = lax.axis_index("c")
        s = lax.axis_index("s")
        t = s * 2 + c

        @pl.when(s == 0)
        def _():
            pltpu.sync_copy(zero_hbm, acc)

        plsc.subcore_barrier()
        pltpu.sync_copy(src_hbm.at[t], src_v)
        pltpu.sync_copy(dst_hbm.at[t], dst_v)
        pltpu.sync_copy(ew_hbm.at[t], ew_v)

        pltpu.async_copy(tbl_hbm.at[src_v.at[0]], gb0, sg0)
        pltpu.async_copy(tbl_hbm.at[src_v.at[1]], gb1, sg1)

        def half(jj, gbuf, sg):
            pltpu.make_async_copy(tbl_hbm.at[src_v.at[jj]], gbuf, sg).wait()

            def scale(eb, cc):
                wv = ew_v[jj, pl.ds(eb * 16, 16)]
                for lane in range(16):
                    e = eb * 16 + lane
                    w = wv[lane]
                    for h in range(width // 16):
                        gbuf[e, pl.ds(h * 16, 16)] = (
                            gbuf[e, pl.ds(h * 16, 16)] * w)
                return cc

            lax.fori_loop(0, CH // 16, scale, 0)
            pltpu.sync_copy(gbuf, acc.at[dst_v.at[jj]], add=True)

            @pl.when(jj + 2 < C_CONV)
            def _():
                pltpu.async_copy(tbl_hbm.at[src_v.at[jj + 2]], gbuf, sg)

        def body(j, carry):
            half(2 * j, gb0, sg0)
            half(2 * j + 1, gb1, sg1)
            return carry

        lax.fori_loop(0, C_CONV // 2, body, 0)
        plsc.subcore_barrier()

        @pl.when(s == 0)
        def _():
            pltpu.sync_copy(acc, out_hbm.at[c])

    return k(tbl, src_idx, dst_idx, ew_rows, zero_w)


def _sc_dots(tbl, a_idx, b_idx):
    @functools.partial(
        pl.kernel,
        out_type=jax.ShapeDtypeStruct((NT, C_DOT, CH, 16), jnp.float32),
        mesh=_sc_mesh(),
        compiler_params=_SC_PARAMS,
        scratch_types=[
            pltpu.VMEM((C_DOT, CH), jnp.int32),
            pltpu.VMEM((C_DOT, CH), jnp.int32),
            pltpu.VMEM((CH, 32), jnp.float32),
            pltpu.VMEM((CH, 32), jnp.float32),
            pltpu.VMEM((CH, 32), jnp.float32),
            pltpu.VMEM((CH, 32), jnp.float32),
            pltpu.VMEM((CH, 16), jnp.float32),
            pltpu.VMEM((CH, 16), jnp.float32),
            pltpu.SemaphoreType.DMA,
            pltpu.SemaphoreType.DMA,
            pltpu.SemaphoreType.DMA,
            pltpu.SemaphoreType.DMA,
        ],
    )
    def k(tbl_hbm, a_hbm, b_hbm, out_hbm, a_v, b_v, ab0, bb0, ab1, bb1,
          pb0, pb1, sg0, sg1, sw0, sw1):
        c = lax.axis_index("c")
        s = lax.axis_index("s")
        t = s * 2 + c
        pltpu.sync_copy(a_hbm.at[t], a_v)
        pltpu.sync_copy(b_hbm.at[t], b_v)

        pltpu.async_copy(tbl_hbm.at[a_v.at[0]], ab0, sg0)
        pltpu.async_copy(tbl_hbm.at[b_v.at[0]], bb0, sg0)
        pltpu.async_copy(tbl_hbm.at[a_v.at[1]], ab1, sg1)
        pltpu.async_copy(tbl_hbm.at[b_v.at[1]], bb1, sg1)

        def half(j, jj, abuf, bbuf, pbuf, sg, sw):
            pltpu.make_async_copy(tbl_hbm.at[a_v.at[jj]], abuf, sg).wait()
            pltpu.make_async_copy(tbl_hbm.at[b_v.at[jj]], bbuf, sg).wait()

            @pl.when(j > 0)
            def _():
                pltpu.make_async_copy(pbuf, out_hbm.at[t, jj], sw).wait()

            def edge(e, cc):
                pbuf[e, pl.ds(0, 16)] = (
                    abuf[e, pl.ds(0, 16)] * bbuf[e, pl.ds(0, 16)]
                    + abuf[e, pl.ds(16, 16)] * bbuf[e, pl.ds(16, 16)])
                return cc

            lax.fori_loop(0, CH, edge, 0, unroll=16)
            pltpu.async_copy(pbuf, out_hbm.at[t, jj], sw)

            @pl.when(jj + 2 < C_DOT)
            def _():
                pltpu.async_copy(tbl_hbm.at[a_v.at[jj + 2]], abuf, sg)
                pltpu.async_copy(tbl_hbm.at[b_v.at[jj + 2]], bbuf, sg)

        def body(j, carry):
            half(j, 2 * j, ab0, bb0, pb0, sg0, sw0)
            half(j, 2 * j + 1, ab1, bb1, pb1, sg1, sw1)
            return carry

        lax.fori_loop(0, C_DOT // 2, body, 0)
        pltpu.make_async_copy(pb0, out_hbm.at[t, C_DOT - 2], sw0).wait()
        pltpu.make_async_copy(pb1, out_hbm.at[t, C_DOT - 1], sw1).wait()

    return k(tbl, a_idx, b_idx)


def _bn_elu(h, g, b):
    m = jnp.mean(h, axis=0)
    v = jnp.mean((h - m) ** 2, axis=0)
    hn = (h - m) * lax.rsqrt(v + 1e-3) * g + b
    return jnp.where(hn > 0, hn, jnp.exp(hn) - 1.0)


def _tc_encoder(x, W1, b1, g1, be1, W2, b2, g2, be2, Wgf, degh):
    def body(x_ref, W1_ref, b1_ref, g1_ref, be1_ref, W2_ref, b2_ref,
             g2_ref, be2_ref, Wgf_ref, degh_ref,
             featx_ref, g1o_ref, gs1_ref, dinv_ref):
        h1 = _bn_elu(
            jnp.dot(x_ref[...], W1_ref[...],
                    preferred_element_type=jnp.float32) + b1_ref[...],
            g1_ref[...], be1_ref[...])
        fx = _bn_elu(
            jnp.dot(h1, W2_ref[...],
                    preferred_element_type=jnp.float32) + b2_ref[...],
            g2_ref[...], be2_ref[...])
        featx_ref[...] = fx
        deg = degh_ref[0] + degh_ref[1] + 1.0
        dinv = lax.rsqrt(deg)
        gv = jnp.dot(fx, Wgf_ref[...], preferred_element_type=jnp.float32)
        g1o_ref[...] = gv
        gs1_ref[...] = gv * dinv
        dinv_ref[...] = dinv

    return pl.pallas_call(
        body,
        out_shape=[
            jax.ShapeDtypeStruct((N, 24), jnp.float32),
            jax.ShapeDtypeStruct((N, 32), jnp.float32),
            jax.ShapeDtypeStruct((N, 32), jnp.float32),
            jax.ShapeDtypeStruct((N, 1), jnp.float32),
        ],
    )(x, W1, b1, g1, be1, W2, b2, g2, be2, Wgf, degh)


def _tc_mid(acc1h, g1m, dinv, b_gf, Wgm, Wgv):
    def body(acc_ref, g1_ref, dinv_ref, bgf_ref, Wgm_ref, Wgv_ref,
             g2o_ref, gs2_ref):
        dinv = dinv_ref[...]
        a = acc_ref[0] + acc_ref[1]
        h = jnp.maximum(
            a * dinv + g1_ref[...] * (dinv * dinv) + bgf_ref[...], 0.0)
        g2m = jnp.dot(h, Wgm_ref[...], preferred_element_type=jnp.float32)
        g2v = jnp.dot(h, Wgv_ref[...], preferred_element_type=jnp.float32)
        g2 = jnp.concatenate([g2m, g2v], axis=1)
        g2o_ref[...] = g2
        gs2_ref[...] = g2 * dinv

    return pl.pallas_call(
        body,
        out_shape=[
            jax.ShapeDtypeStruct((N, 16), jnp.float32),
            jax.ShapeDtypeStruct((N, 16), jnp.float32),
        ],
    )(acc1h, g1m, dinv, b_gf, Wgm, Wgv)


def _tc_final(acc2h, g2m, dinv, b_gm, b_gv, featx, Wd, bd, gd, bed, x):
    def body(acc_ref, g2_ref, dinv_ref, bgm_ref, bgv_ref, featx_ref,
             Wd_ref, bd_ref, gd_ref, bed_ref, x_ref,
             feat_ref, dae_ref, kl_ref):
        dinv = dinv_ref[...]
        tt = (acc_ref[0] + acc_ref[1]) * dinv + g2_ref[...] * (dinv * dinv)
        mu = tt[:, :8] + bgm_ref[...]
        ls = jnp.minimum(tt[:, 8:] + bgv_ref[...], 10.0)
        feat = jnp.concatenate([featx_ref[...], mu], axis=1)
        feat_ref[...] = feat
        xd = _bn_elu(
            jnp.dot(feat, Wd_ref[...],
                    preferred_element_type=jnp.float32) + bd_ref[...],
            gd_ref[...], bed_ref[...])
        dae_ref[...] = jnp.reshape(jnp.mean((xd - x_ref[...]) ** 2), (1, 1))
        kl_ref[...] = jnp.reshape(-0.5 * jnp.mean(
            jnp.sum(1.0 + 2.0 * ls - mu * mu - jnp.exp(2.0 * ls), axis=1)),
            (1, 1))

    return pl.pallas_call(
        body,
        out_shape=[
            jax.ShapeDtypeStruct((N, 32), jnp.float32),
            jax.ShapeDtypeStruct((1, 1), jnp.float32),
            jax.ShapeDtypeStruct((1, 1), jnp.float32),
        ],
    )(acc2h, g2m, dinv, b_gm, b_gv, featx, Wd, bd, gd, bed, x)


def _tc_loss(pos_pp, neg_pp, ew8, kl):
    def body(pos_ref, neg_ref, ew_ref, kl_ref, out_ref):
        seg = (lax.broadcasted_iota(jnp.int32, (128, 8), 0) // 16
               == lax.broadcasted_iota(jnp.int32, (128, 8), 1)
               ).astype(jnp.float32)
        praw = jnp.dot(pos_ref[...], seg, preferred_element_type=jnp.float32)
        p = 1.0 / (1.0 + jnp.exp(-praw))
        pos_loss = jnp.mean(p * (1.0 - ew_ref[...])
                            + jnp.log(1.0 + jnp.exp(-p)))
        qraw = jnp.dot(neg_ref[...], seg, preferred_element_type=jnp.float32)
        q = 1.0 / (1.0 + jnp.exp(-qraw))
        neg_loss = jnp.mean(jnp.log(1.0 + jnp.exp(q)))
        out_ref[...] = (pos_loss + neg_loss) + kl_ref[...] / N

    return pl.pallas_call(
        body,
        out_shape=jax.ShapeDtypeStruct((1, 1), jnp.float32),
    )(pos_pp, neg_pp, ew8, kl)


def kernel(x, edge_index, edge_weight, W_enc1, b_enc1, g_enc1, be_enc1,
           W_enc2, b_enc2, g_enc2, be_enc2, W_gf, b_gf, W_gm, b_gm,
           W_gv, b_gv, W_dec, b_dec, g_dec, be_dec):
    src = edge_index[0].astype(jnp.int32)
    dst = edge_index[1].astype(jnp.int32)
    ew = edge_weight.astype(jnp.float32)

    pad = EP - E
    zpad_i = jnp.zeros((pad,), jnp.int32)
    srcp = jnp.concatenate([src, zpad_i]).reshape(NT, C_CONV, CH)
    dstp = jnp.concatenate([dst, zpad_i]).reshape(NT, C_CONV, CH)
    ewp = jnp.concatenate([ew, jnp.zeros((pad,), jnp.float32)]
                          ).reshape(NT, C_CONV, CH)
    zero1 = jnp.zeros((N,), jnp.float32)
    zero32 = jnp.zeros((N, 32), jnp.float32)
    zero16 = jnp.zeros((N, 16), jnp.float32)

    degh = _sc_deg(ewp, dstp, zero1)
    featx, g1m, gs1, dinv = _tc_encoder(
        x, W_enc1, b_enc1, g_enc1, be_enc1,
        W_enc2, b_enc2, g_enc2, be_enc2, W_gf, degh.reshape(2, N, 1))
    acc1 = _sc_conv(gs1, srcp, dstp, ewp, zero32, 32)
    g2m, gs2 = _tc_mid(acc1, g1m, dinv, b_gf, W_gm, W_gv)
    acc2 = _sc_conv(gs2, srcp, dstp, ewp, zero16, 16)
    feat, dae, kl = _tc_final(
        acc2, g2m, dinv, b_gm, b_gv, featx, W_dec, b_dec, g_dec, be_dec, x)

    neg = jax.random.randint(jax.random.key(42), (2, E), 0, N
                             ).astype(jnp.int32)
    pad2 = EP2 - 2 * E
    zpad2 = jnp.zeros((pad2,), jnp.int32)
    a_all = jnp.concatenate([src, neg[0], zpad2]).reshape(NT, C_DOT, CH)
    b_all = jnp.concatenate([dst, neg[1], zpad2]).reshape(NT, C_DOT, CH)
    dots = _sc_dots(feat, a_all, b_all).reshape(EP2 * 16)
    pos_pp = dots[:E * 16].reshape(E // 8, 128)
    neg_pp = dots[E * 16:2 * E * 16].reshape(E // 8, 128)
    gae = _tc_loss(pos_pp, neg_pp, ew.reshape(E // 8, 8), kl)

    return (feat, dae.reshape(()), gae.reshape(()))


# speedup vs baseline: 1.0585x; 1.0585x over previous
import functools

import jax
import jax.numpy as jnp
from jax import lax
from jax.experimental import pallas as pl
from jax.experimental.pallas import tpu as pltpu
from jax.experimental.pallas import tpu_sc as plsc

N = 10000
E = 160000
NT = 32
CH = 128
C_CONV = 40
EP = NT * C_CONV * CH
C_DOT = 80
EP2 = NT * C_DOT * CH


def _sc_mesh():
    return plsc.VectorSubcoreMesh(core_axis_name="c", subcore_axis_name="s")


_SC_PARAMS = pltpu.CompilerParams(
    use_tc_tiling_on_sc=False, needs_layout_passes=False)


def _sc_deg(ew_rows, dst_idx, zero1):
    @functools.partial(
        pl.kernel,
        out_type=jax.ShapeDtypeStruct((2, N), jnp.float32),
        mesh=_sc_mesh(),
        compiler_params=_SC_PARAMS,
        scratch_types=[
            pltpu.VMEM((C_CONV, CH), jnp.float32),
            pltpu.VMEM((C_CONV, CH), jnp.int32),
            pltpu.VMEM_SHARED((N,), jnp.float32),
        ],
    )
    def k(ew_hbm, dst_hbm, zero_hbm, out_hbm, ew_v, dst_v, acc):
        c = lax.axis_index("c")
        s = lax.axis_index("s")
        t = s * 2 + c

        @pl.when(s == 0)
        def _():
            pltpu.sync_copy(zero_hbm, acc)

        plsc.subcore_barrier()
        pltpu.sync_copy(ew_hbm.at[t], ew_v)
        pltpu.sync_copy(dst_hbm.at[t], dst_v)

        def body(j, carry):
            pltpu.sync_copy(ew_v.at[j], acc.at[dst_v.at[j]], add=True)
            return carry

        lax.fori_loop(0, C_CONV, body, 0)
        plsc.subcore_barrier()

        @pl.when(s == 0)
        def _():
            pltpu.sync_copy(acc, out_hbm.at[c])

    return k(ew_rows, dst_idx, zero1)


def _sc_conv(tbl, src_idx, dst_idx, ew_rows, zero_w, width):
    @functools.partial(
        pl.kernel,
        out_type=jax.ShapeDtypeStruct((2, N, width), jnp.float32),
        mesh=_sc_mesh(),
        compiler_params=_SC_PARAMS,
        scratch_types=[
            pltpu.VMEM((C_CONV, CH), jnp.int32),
            pltpu.VMEM((C_CONV, CH), jnp.int32),
            pltpu.VMEM((C_CONV, CH), jnp.float32),
            pltpu.VMEM((CH, width), jnp.float32),
            pltpu.VMEM((CH, width), jnp.float32),
            pltpu.VMEM_SHARED((N, width), jnp.float32),
            pltpu.SemaphoreType.DMA,
            pltpu.SemaphoreType.DMA,
        ],
    )
    def k(tbl_hbm, src_hbm, dst_hbm, ew_hbm, zero_hbm, out_hbm,
          src_v, dst_v, ew_v, gb0, gb1, acc, sg0, sg1):
        c = lax.axis_index("c")
        s = lax.axis_index("s")
        t = s * 2 + c

        @pl.when(s == 0)
        def _():
            pltpu.sync_copy(zero_hbm, acc)

        plsc.subcore_barrier()
        pltpu.sync_copy(src_hbm.at[t], src_v)
        pltpu.sync_copy(dst_hbm.at[t], dst_v)
        pltpu.sync_copy(ew_hbm.at[t], ew_v)

        pltpu.async_copy(tbl_hbm.at[src_v.at[0]], gb0, sg0)
        pltpu.async_copy(tbl_hbm.at[src_v.at[1]], gb1, sg1)

        def half(jj, gbuf, sg):
            pltpu.make_async_copy(tbl_hbm.at[src_v.at[jj]], gbuf, sg).wait()

            def scale(eb, cc):
                wv = ew_v[jj, pl.ds(eb * 16, 16)]
                for lane in range(16):
                    e = eb * 16 + lane
                    w = wv[lane]
                    for h in range(width // 16):
                        gbuf[e, pl.ds(h * 16, 16)] = (
                            gbuf[e, pl.ds(h * 16, 16)] * w)
                return cc

            lax.fori_loop(0, CH // 16, scale, 0)
            pltpu.sync_copy(gbuf, acc.at[dst_v.at[jj]], add=True)

            @pl.when(jj + 2 < C_CONV)
            def _():
                pltpu.async_copy(tbl_hbm.at[src_v.at[jj + 2]], gbuf, sg)

        def body(j, carry):
            half(2 * j, gb0, sg0)
            half(2 * j + 1, gb1, sg1)
            return carry

        lax.fori_loop(0, C_CONV // 2, body, 0)
        plsc.subcore_barrier()

        @pl.when(s == 0)
        def _():
            pltpu.sync_copy(acc, out_hbm.at[c])

    return k(tbl, src_idx, dst_idx, ew_rows, zero_w)


C_TOT = NT * C_CONV
CS = 25
CF = C_CONV * 2 - CS
SLOW_CORE = 0


def _sc_dots(tbl, pa_idx, pb_idx, na_idx, nb_idx):
    @functools.partial(
        pl.kernel,
        out_type=[
            jax.ShapeDtypeStruct((C_TOT, CH, 16), jnp.float32),
            jax.ShapeDtypeStruct((C_TOT, CH, 16), jnp.float32),
        ],
        mesh=_sc_mesh(),
        compiler_params=_SC_PARAMS,
        scratch_types=[
            pltpu.VMEM((CF, CH), jnp.int32),
            pltpu.VMEM((CF, CH), jnp.int32),
            pltpu.VMEM((CF, CH), jnp.int32),
            pltpu.VMEM((CF, CH), jnp.int32),
            pltpu.VMEM((CH, 32), jnp.float32),
            pltpu.VMEM((CH, 32), jnp.float32),
            pltpu.VMEM((CH, 32), jnp.float32),
            pltpu.VMEM((CH, 32), jnp.float32),
            pltpu.VMEM((CH, 16), jnp.float32),
            pltpu.VMEM((CH, 16), jnp.float32),
            pltpu.SemaphoreType.DMA,
            pltpu.SemaphoreType.DMA,
            pltpu.SemaphoreType.DMA,
            pltpu.SemaphoreType.DMA,
        ],
    )
    def k(tbl_hbm, pa_hbm, pb_hbm, na_hbm, nb_hbm, opos_hbm, oneg_hbm,
          pav, pbv, nav, nbv, abA, bbA, abB, bbB, pbA, pbB,
          sgA, sgB, swA, swB):
        c = lax.axis_index("c")
        s = lax.axis_index("s")

        def run(lo, cnt):
            pltpu.sync_copy(pa_hbm.at[pl.ds(lo, cnt)], pav.at[pl.ds(0, cnt)])
            pltpu.sync_copy(pb_hbm.at[pl.ds(lo, cnt)], pbv.at[pl.ds(0, cnt)])
            pltpu.sync_copy(na_hbm.at[pl.ds(lo, cnt)], nav.at[pl.ds(0, cnt)])
            pltpu.sync_copy(nb_hbm.at[pl.ds(lo, cnt)], nbv.at[pl.ds(0, cnt)])

            pltpu.async_copy(tbl_hbm.at[pav.at[0]], abA, sgA)
            pltpu.async_copy(tbl_hbm.at[pbv.at[0]], bbA, sgA)
            pltpu.async_copy(tbl_hbm.at[nav.at[0]], abB, sgB)
            pltpu.async_copy(tbl_hbm.at[nbv.at[0]], bbB, sgB)

            def half(j, av, bv, abuf, bbuf, pbuf, sg, sw, out_hbm):
                pltpu.make_async_copy(tbl_hbm.at[av.at[j]], abuf, sg).wait()
                pltpu.make_async_copy(tbl_hbm.at[bv.at[j]], bbuf, sg).wait()

                @pl.when(j > 0)
                def _():
                    pltpu.make_async_copy(
                        pbuf, out_hbm.at[lo + j], sw).wait()

                def edge(e, cc):
                    pbuf[e, pl.ds(0, 16)] = (
                        abuf[e, pl.ds(0, 16)] * bbuf[e, pl.ds(0, 16)]
                        + abuf[e, pl.ds(16, 16)] * bbuf[e, pl.ds(16, 16)])
                    return cc

                lax.fori_loop(0, CH, edge, 0, unroll=16)
                pltpu.async_copy(pbuf, out_hbm.at[lo + j], sw)

                @pl.when(j + 1 < cnt)
                def _():
                    pltpu.async_copy(tbl_hbm.at[av.at[j + 1]], abuf, sg)
                    pltpu.async_copy(tbl_hbm.at[bv.at[j + 1]], bbuf, sg)

            def body(j, carry):
                half(j, pav, pbv, abA, bbA, pbA, sgA, swA, opos_hbm)
                half(j, nav, nbv, abB, bbB, pbB, sgB, swB, oneg_hbm)
                return carry

            lax.fori_loop(0, cnt, body, 0)
            pltpu.make_async_copy(pbA, opos_hbm.at[lo + cnt - 1], swA).wait()
            pltpu.make_async_copy(pbB, oneg_hbm.at[lo + cnt - 1], swB).wait()

        @pl.when(c == SLOW_CORE)
        def _():
            run(s * CS, CS)

        @pl.when(c != SLOW_CORE)
        def _():
            run(16 * CS + s * CF, CF)

    return k(tbl, pa_idx, pb_idx, na_idx, nb_idx)


def _bn_elu(h, g, b):
    m = jnp.mean(h, axis=0)
    v = jnp.mean((h - m) ** 2, axis=0)
    hn = (h - m) * lax.rsqrt(v + 1e-3) * g + b
    return jnp.where(hn > 0, hn, jnp.exp(hn) - 1.0)


def _tc_encoder(x, W1, b1, g1, be1, W2, b2, g2, be2, Wgf, degh):
    def body(x_ref, W1_ref, b1_ref, g1_ref, be1_ref, W2_ref, b2_ref,
             g2_ref, be2_ref, Wgf_ref, degh_ref,
             featx_ref, g1o_ref, gs1_ref, dinv_ref):
        h1 = _bn_elu(
            jnp.dot(x_ref[...], W1_ref[...],
                    preferred_element_type=jnp.float32) + b1_ref[...],
            g1_ref[...], be1_ref[...])
        fx = _bn_elu(
            jnp.dot(h1, W2_ref[...],
                    preferred_element_type=jnp.float32) + b2_ref[...],
            g2_ref[...], be2_ref[...])
        featx_ref[...] = fx
        deg = degh_ref[0] + degh_ref[1] + 1.0
        dinv = lax.rsqrt(deg)
        gv = jnp.dot(fx, Wgf_ref[...], preferred_element_type=jnp.float32)
        g1o_ref[...] = gv
        gs1_ref[...] = gv * dinv
        dinv_ref[...] = dinv

    return pl.pallas_call(
        body,
        out_shape=[
            jax.ShapeDtypeStruct((N, 24), jnp.float32),
            jax.ShapeDtypeStruct((N, 32), jnp.float32),
            jax.ShapeDtypeStruct((N, 32), jnp.float32),
            jax.ShapeDtypeStruct((N, 1), jnp.float32),
        ],
    )(x, W1, b1, g1, be1, W2, b2, g2, be2, Wgf, degh)


def _tc_mid(acc1h, g1m, dinv, b_gf, Wgm, Wgv):
    def body(acc_ref, g1_ref, dinv_ref, bgf_ref, Wgm_ref, Wgv_ref,
             g2o_ref, gs2_ref):
        dinv = dinv_ref[...]
        a = acc_ref[0] + acc_ref[1]
        h = jnp.maximum(
            a * dinv + g1_ref[...] * (dinv * dinv) + bgf_ref[...], 0.0)
        g2m = jnp.dot(h, Wgm_ref[...], preferred_element_type=jnp.float32)
        g2v = jnp.dot(h, Wgv_ref[...], preferred_element_type=jnp.float32)
        g2 = jnp.concatenate([g2m, g2v], axis=1)
        g2o_ref[...] = g2
        gs2_ref[...] = g2 * dinv

    return pl.pallas_call(
        body,
        out_shape=[
            jax.ShapeDtypeStruct((N, 16), jnp.float32),
            jax.ShapeDtypeStruct((N, 16), jnp.float32),
        ],
    )(acc1h, g1m, dinv, b_gf, Wgm, Wgv)


def _tc_final(acc2h, g2m, dinv, b_gm, b_gv, featx, Wd, bd, gd, bed, x):
    def body(acc_ref, g2_ref, dinv_ref, bgm_ref, bgv_ref, featx_ref,
             Wd_ref, bd_ref, gd_ref, bed_ref, x_ref,
             feat_ref, dae_ref, kl_ref):
        dinv = dinv_ref[...]
        tt = (acc_ref[0] + acc_ref[1]) * dinv + g2_ref[...] * (dinv * dinv)
        mu = tt[:, :8] + bgm_ref[...]
        ls = jnp.minimum(tt[:, 8:] + bgv_ref[...], 10.0)
        feat = jnp.concatenate([featx_ref[...], mu], axis=1)
        feat_ref[...] = feat
        xd = _bn_elu(
            jnp.dot(feat, Wd_ref[...],
                    preferred_element_type=jnp.float32) + bd_ref[...],
            gd_ref[...], bed_ref[...])
        dae_ref[...] = jnp.reshape(jnp.mean((xd - x_ref[...]) ** 2), (1, 1))
        kl_ref[...] = jnp.reshape(-0.5 * jnp.mean(
            jnp.sum(1.0 + 2.0 * ls - mu * mu - jnp.exp(2.0 * ls), axis=1)),
            (1, 1))

    return pl.pallas_call(
        body,
        out_shape=[
            jax.ShapeDtypeStruct((N, 32), jnp.float32),
            jax.ShapeDtypeStruct((1, 1), jnp.float32),
            jax.ShapeDtypeStruct((1, 1), jnp.float32),
        ],
    )(acc2h, g2m, dinv, b_gm, b_gv, featx, Wd, bd, gd, bed, x)


def _tc_loss(pos_pp, neg_pp, ew8, kl):
    rows = EP // 8
    vrows = E // 8

    def body(pos_ref, neg_ref, ew_ref, kl_ref, out_ref):
        seg = (lax.broadcasted_iota(jnp.int32, (128, 8), 0) // 16
               == lax.broadcasted_iota(jnp.int32, (128, 8), 1)
               ).astype(jnp.float32)
        valid = lax.broadcasted_iota(jnp.int32, (rows, 8), 0) < vrows
        praw = jnp.dot(pos_ref[...], seg, preferred_element_type=jnp.float32)
        p = 1.0 / (1.0 + jnp.exp(-praw))
        pos_t = p * (1.0 - ew_ref[...]) + jnp.log(1.0 + jnp.exp(-p))
        pos_loss = jnp.sum(jnp.where(valid, pos_t, 0.0)) / E
        qraw = jnp.dot(neg_ref[...], seg, preferred_element_type=jnp.float32)
        q = 1.0 / (1.0 + jnp.exp(-qraw))
        neg_t = jnp.log(1.0 + jnp.exp(q))
        neg_loss = jnp.sum(jnp.where(valid, neg_t, 0.0)) / E
        out_ref[...] = (pos_loss + neg_loss) + kl_ref[...] / N

    return pl.pallas_call(
        body,
        out_shape=jax.ShapeDtypeStruct((1, 1), jnp.float32),
    )(pos_pp, neg_pp, ew8, kl)


def kernel(x, edge_index, edge_weight, W_enc1, b_enc1, g_enc1, be_enc1,
           W_enc2, b_enc2, g_enc2, be_enc2, W_gf, b_gf, W_gm, b_gm,
           W_gv, b_gv, W_dec, b_dec, g_dec, be_dec):
    src = edge_index[0].astype(jnp.int32)
    dst = edge_index[1].astype(jnp.int32)
    ew = edge_weight.astype(jnp.float32)

    pad = EP - E
    zpad_i = jnp.zeros((pad,), jnp.int32)
    srcp = jnp.concatenate([src, zpad_i]).reshape(NT, C_CONV, CH)
    dstp = jnp.concatenate([dst, zpad_i]).reshape(NT, C_CONV, CH)
    ewp = jnp.concatenate([ew, jnp.zeros((pad,), jnp.float32)]
                          ).reshape(NT, C_CONV, CH)
    zero1 = jnp.zeros((N,), jnp.float32)
    zero32 = jnp.zeros((N, 32), jnp.float32)
    zero16 = jnp.zeros((N, 16), jnp.float32)

    degh = _sc_deg(ewp, dstp, zero1)
    featx, g1m, gs1, dinv = _tc_encoder(
        x, W_enc1, b_enc1, g_enc1, be_enc1,
        W_enc2, b_enc2, g_enc2, be_enc2, W_gf, degh.reshape(2, N, 1))
    acc1 = _sc_conv(gs1, srcp, dstp, ewp, zero32, 32)
    g2m, gs2 = _tc_mid(acc1, g1m, dinv, b_gf, W_gm, W_gv)
    acc2 = _sc_conv(gs2, srcp, dstp, ewp, zero16, 16)
    feat, dae, kl = _tc_final(
        acc2, g2m, dinv, b_gm, b_gv, featx, W_dec, b_dec, g_dec, be_dec, x)

    neg = jax.random.randint(jax.random.key(42), (2, E), 0, N
                             ).astype(jnp.int32)
    nega = jnp.concatenate([neg[0], zpad_i]).reshape(C_TOT, CH)
    negb = jnp.concatenate([neg[1], zpad_i]).reshape(C_TOT, CH)
    out_pos, out_neg = _sc_dots(
        feat, srcp.reshape(C_TOT, CH), dstp.reshape(C_TOT, CH), nega, negb)
    pos_pp = out_pos.reshape(EP // 8, 128)
    neg_pp = out_neg.reshape(EP // 8, 128)
    gae = _tc_loss(pos_pp, neg_pp, ewp.reshape(EP // 8, 8), kl)

    return (feat, dae.reshape(()), gae.reshape(()))


# speedup vs baseline: 1.0782x; 1.0186x over previous
import functools

import jax
import jax.numpy as jnp
from jax import lax
from jax.experimental import pallas as pl
from jax.experimental.pallas import tpu as pltpu
from jax.experimental.pallas import tpu_sc as plsc

N = 10000
E = 160000
NT = 32
CH = 128
C_CONV = 40
EP = NT * C_CONV * CH
C_DOT = 80
EP2 = NT * C_DOT * CH


def _sc_mesh():
    return plsc.VectorSubcoreMesh(core_axis_name="c", subcore_axis_name="s")


_SC_PARAMS = pltpu.CompilerParams(
    use_tc_tiling_on_sc=False, needs_layout_passes=False)


def _sc_deg(ew_rows, dst_idx, zero1):
    @functools.partial(
        pl.kernel,
        out_type=jax.ShapeDtypeStruct((2, N), jnp.float32),
        mesh=_sc_mesh(),
        compiler_params=_SC_PARAMS,
        scratch_types=[
            pltpu.VMEM((C_CONV, CH), jnp.float32),
            pltpu.VMEM((C_CONV, CH), jnp.int32),
            pltpu.VMEM_SHARED((N,), jnp.float32),
        ],
    )
    def k(ew_hbm, dst_hbm, zero_hbm, out_hbm, ew_v, dst_v, acc):
        c = lax.axis_index("c")
        s = lax.axis_index("s")
        t = s * 2 + c

        @pl.when(s == 0)
        def _():
            pltpu.sync_copy(zero_hbm, acc)

        plsc.subcore_barrier()
        pltpu.sync_copy(ew_hbm.at[t], ew_v)
        pltpu.sync_copy(dst_hbm.at[t], dst_v)

        def body(j, carry):
            pltpu.sync_copy(ew_v.at[j], acc.at[dst_v.at[j]], add=True)
            return carry

        lax.fori_loop(0, C_CONV, body, 0)
        plsc.subcore_barrier()

        @pl.when(s == 0)
        def _():
            pltpu.sync_copy(acc, out_hbm.at[c])

    return k(ew_rows, dst_idx, zero1)


def _sc_conv(tbl, src_idx, dst_idx, ew_rows, zero_w, width):
    @functools.partial(
        pl.kernel,
        out_type=jax.ShapeDtypeStruct((2, N, width), jnp.float32),
        mesh=_sc_mesh(),
        compiler_params=_SC_PARAMS,
        scratch_types=[
            pltpu.VMEM((C_CONV, CH), jnp.int32),
            pltpu.VMEM((C_CONV, CH), jnp.int32),
            pltpu.VMEM((C_CONV, CH), jnp.float32),
            pltpu.VMEM((CH, width), jnp.float32),
            pltpu.VMEM((CH, width), jnp.float32),
            pltpu.VMEM_SHARED((N, width), jnp.float32),
            pltpu.SemaphoreType.DMA,
            pltpu.SemaphoreType.DMA,
        ],
    )
    def k(tbl_hbm, src_hbm, dst_hbm, ew_hbm, zero_hbm, out_hbm,
          src_v, dst_v, ew_v, gb0, gb1, acc, sg0, sg1):
        c = lax.axis_index("c")
        s = lax.axis_index("s")
        t = s * 2 + c

        @pl.when(s == 0)
        def _():
            pltpu.sync_copy(zero_hbm, acc)

        plsc.subcore_barrier()
        pltpu.sync_copy(src_hbm.at[t], src_v)
        pltpu.sync_copy(dst_hbm.at[t], dst_v)
        pltpu.sync_copy(ew_hbm.at[t], ew_v)

        pltpu.async_copy(tbl_hbm.at[src_v.at[0]], gb0, sg0)
        pltpu.async_copy(tbl_hbm.at[src_v.at[1]], gb1, sg1)

        def half(jj, gbuf, sg):
            pltpu.make_async_copy(tbl_hbm.at[src_v.at[jj]], gbuf, sg).wait()

            def scale(eb, cc):
                wv = ew_v[jj, pl.ds(eb * 16, 16)]
                for lane in range(16):
                    e = eb * 16 + lane
                    w = wv[lane]
                    for h in range(width // 16):
                        gbuf[e, pl.ds(h * 16, 16)] = (
                            gbuf[e, pl.ds(h * 16, 16)] * w)
                return cc

            lax.fori_loop(0, CH // 16, scale, 0)
            pltpu.sync_copy(gbuf, acc.at[dst_v.at[jj]], add=True)

            @pl.when(jj + 2 < C_CONV)
            def _():
                pltpu.async_copy(tbl_hbm.at[src_v.at[jj + 2]], gbuf, sg)

        def body(j, carry):
            half(2 * j, gb0, sg0)
            half(2 * j + 1, gb1, sg1)
            return carry

        lax.fori_loop(0, C_CONV // 2, body, 0)
        plsc.subcore_barrier()

        @pl.when(s == 0)
        def _():
            pltpu.sync_copy(acc, out_hbm.at[c])

    return k(tbl, src_idx, dst_idx, ew_rows, zero_w)


C_TOT = NT * C_CONV
CS = 25
CF = C_CONV * 2 - CS
SLOW_CORE = 1


def _sc_dots(tbl, pa_idx, pb_idx, na_idx, nb_idx):
    @functools.partial(
        pl.kernel,
        out_type=[
            jax.ShapeDtypeStruct((C_TOT, CH, 16), jnp.float32),
            jax.ShapeDtypeStruct((C_TOT, CH, 16), jnp.float32),
        ],
        mesh=_sc_mesh(),
        compiler_params=_SC_PARAMS,
        scratch_types=[
            pltpu.VMEM((CF, CH), jnp.int32),
            pltpu.VMEM((CF, CH), jnp.int32),
            pltpu.VMEM((CF, CH), jnp.int32),
            pltpu.VMEM((CF, CH), jnp.int32),
            pltpu.VMEM((CH, 32), jnp.float32),
            pltpu.VMEM((CH, 32), jnp.float32),
            pltpu.VMEM((CH, 32), jnp.float32),
            pltpu.VMEM((CH, 32), jnp.float32),
            pltpu.VMEM((CH, 16), jnp.float32),
            pltpu.VMEM((CH, 16), jnp.float32),
            pltpu.SemaphoreType.DMA,
            pltpu.SemaphoreType.DMA,
            pltpu.SemaphoreType.DMA,
            pltpu.SemaphoreType.DMA,
        ],
    )
    def k(tbl_hbm, pa_hbm, pb_hbm, na_hbm, nb_hbm, opos_hbm, oneg_hbm,
          pav, pbv, nav, nbv, abA, bbA, abB, bbB, pbA, pbB,
          sgA, sgB, swA, swB):
        c = lax.axis_index("c")
        s = lax.axis_index("s")

        def run(lo, cnt):
            pltpu.sync_copy(pa_hbm.at[pl.ds(lo, cnt)], pav.at[pl.ds(0, cnt)])
            pltpu.sync_copy(pb_hbm.at[pl.ds(lo, cnt)], pbv.at[pl.ds(0, cnt)])
            pltpu.sync_copy(na_hbm.at[pl.ds(lo, cnt)], nav.at[pl.ds(0, cnt)])
            pltpu.sync_copy(nb_hbm.at[pl.ds(lo, cnt)], nbv.at[pl.ds(0, cnt)])

            pltpu.async_copy(tbl_hbm.at[pav.at[0]], abA, sgA)
            pltpu.async_copy(tbl_hbm.at[pbv.at[0]], bbA, sgA)
            pltpu.async_copy(tbl_hbm.at[nav.at[0]], abB, sgB)
            pltpu.async_copy(tbl_hbm.at[nbv.at[0]], bbB, sgB)

            def half(j, av, bv, abuf, bbuf, pbuf, sg, sw, out_hbm):
                pltpu.make_async_copy(tbl_hbm.at[av.at[j]], abuf, sg).wait()
                pltpu.make_async_copy(tbl_hbm.at[bv.at[j]], bbuf, sg).wait()

                @pl.when(j > 0)
                def _():
                    pltpu.make_async_copy(
                        pbuf, out_hbm.at[lo + j], sw).wait()

                def edge(e, cc):
                    pbuf[e, pl.ds(0, 16)] = (
                        abuf[e, pl.ds(0, 16)] * bbuf[e, pl.ds(0, 16)]
                        + abuf[e, pl.ds(16, 16)] * bbuf[e, pl.ds(16, 16)])
                    return cc

                lax.fori_loop(0, CH, edge, 0, unroll=16)
                pltpu.async_copy(pbuf, out_hbm.at[lo + j], sw)

                @pl.when(j + 1 < cnt)
                def _():
                    pltpu.async_copy(tbl_hbm.at[av.at[j + 1]], abuf, sg)
                    pltpu.async_copy(tbl_hbm.at[bv.at[j + 1]], bbuf, sg)

            def body(j, carry):
                half(j, pav, pbv, abA, bbA, pbA, sgA, swA, opos_hbm)
                half(j, nav, nbv, abB, bbB, pbB, sgB, swB, oneg_hbm)
                return carry

            lax.fori_loop(0, cnt, body, 0)
            pltpu.make_async_copy(pbA, opos_hbm.at[lo + cnt - 1], swA).wait()
            pltpu.make_async_copy(pbB, oneg_hbm.at[lo + cnt - 1], swB).wait()

        @pl.when(c == SLOW_CORE)
        def _():
            run(s * CS, CS)

        @pl.when(c != SLOW_CORE)
        def _():
            run(16 * CS + s * CF, CF)

    return k(tbl, pa_idx, pb_idx, na_idx, nb_idx)


def _bn_elu(h, g, b):
    m = jnp.mean(h, axis=0)
    v = jnp.mean((h - m) ** 2, axis=0)
    hn = (h - m) * lax.rsqrt(v + 1e-3) * g + b
    return jnp.where(hn > 0, hn, jnp.exp(hn) - 1.0)


def _tc_encoder(x, W1, b1, g1, be1, W2, b2, g2, be2, Wgf, degh):
    def body(x_ref, W1_ref, b1_ref, g1_ref, be1_ref, W2_ref, b2_ref,
             g2_ref, be2_ref, Wgf_ref, degh_ref,
             featx_ref, g1o_ref, gs1_ref, dinv_ref):
        h1 = _bn_elu(
            jnp.dot(x_ref[...], W1_ref[...],
                    preferred_element_type=jnp.float32) + b1_ref[...],
            g1_ref[...], be1_ref[...])
        fx = _bn_elu(
            jnp.dot(h1, W2_ref[...],
                    preferred_element_type=jnp.float32) + b2_ref[...],
            g2_ref[...], be2_ref[...])
        featx_ref[...] = fx
        deg = degh_ref[0] + degh_ref[1] + 1.0
        dinv = lax.rsqrt(deg)
        gv = jnp.dot(fx, Wgf_ref[...], preferred_element_type=jnp.float32)
        g1o_ref[...] = gv
        gs1_ref[...] = gv * dinv
        dinv_ref[...] = dinv

    return pl.pallas_call(
        body,
        out_shape=[
            jax.ShapeDtypeStruct((N, 24), jnp.float32),
            jax.ShapeDtypeStruct((N, 32), jnp.float32),
            jax.ShapeDtypeStruct((N, 32), jnp.float32),
            jax.ShapeDtypeStruct((N, 1), jnp.float32),
        ],
    )(x, W1, b1, g1, be1, W2, b2, g2, be2, Wgf, degh)


def _tc_mid(acc1h, g1m, dinv, b_gf, Wgm, Wgv):
    def body(acc_ref, g1_ref, dinv_ref, bgf_ref, Wgm_ref, Wgv_ref,
             g2o_ref, gs2_ref):
        dinv = dinv_ref[...]
        a = acc_ref[0] + acc_ref[1]
        h = jnp.maximum(
            a * dinv + g1_ref[...] * (dinv * dinv) + bgf_ref[...], 0.0)
        g2m = jnp.dot(h, Wgm_ref[...], preferred_element_type=jnp.float32)
        g2v = jnp.dot(h, Wgv_ref[...], preferred_element_type=jnp.float32)
        g2 = jnp.concatenate([g2m, g2v], axis=1)
        g2o_ref[...] = g2
        gs2_ref[...] = g2 * dinv

    return pl.pallas_call(
        body,
        out_shape=[
            jax.ShapeDtypeStruct((N, 16), jnp.float32),
            jax.ShapeDtypeStruct((N, 16), jnp.float32),
        ],
    )(acc1h, g1m, dinv, b_gf, Wgm, Wgv)


def _tc_final(acc2h, g2m, dinv, b_gm, b_gv, featx, Wd, bd, gd, bed, x):
    def body(acc_ref, g2_ref, dinv_ref, bgm_ref, bgv_ref, featx_ref,
             Wd_ref, bd_ref, gd_ref, bed_ref, x_ref,
             feat_ref, dae_ref, kl_ref):
        dinv = dinv_ref[...]
        tt = (acc_ref[0] + acc_ref[1]) * dinv + g2_ref[...] * (dinv * dinv)
        mu = tt[:, :8] + bgm_ref[...]
        ls = jnp.minimum(tt[:, 8:] + bgv_ref[...], 10.0)
        feat = jnp.concatenate([featx_ref[...], mu], axis=1)
        feat_ref[...] = feat
        xd = _bn_elu(
            jnp.dot(feat, Wd_ref[...],
                    preferred_element_type=jnp.float32) + bd_ref[...],
            gd_ref[...], bed_ref[...])
        dae_ref[...] = jnp.reshape(jnp.mean((xd - x_ref[...]) ** 2), (1, 1))
        kl_ref[...] = jnp.reshape(-0.5 * jnp.mean(
            jnp.sum(1.0 + 2.0 * ls - mu * mu - jnp.exp(2.0 * ls), axis=1)),
            (1, 1))

    return pl.pallas_call(
        body,
        out_shape=[
            jax.ShapeDtypeStruct((N, 32), jnp.float32),
            jax.ShapeDtypeStruct((1, 1), jnp.float32),
            jax.ShapeDtypeStruct((1, 1), jnp.float32),
        ],
    )(acc2h, g2m, dinv, b_gm, b_gv, featx, Wd, bd, gd, bed, x)


def _tc_loss(pos_pp, neg_pp, ew8, kl):
    rows = EP // 8
    vrows = E // 8

    def body(pos_ref, neg_ref, ew_ref, kl_ref, out_ref):
        seg = (lax.broadcasted_iota(jnp.int32, (128, 8), 0) // 16
               == lax.broadcasted_iota(jnp.int32, (128, 8), 1)
               ).astype(jnp.float32)
        valid = lax.broadcasted_iota(jnp.int32, (rows, 8), 0) < vrows
        praw = jnp.dot(pos_ref[...], seg, preferred_element_type=jnp.float32)
        p = 1.0 / (1.0 + jnp.exp(-praw))
        pos_t = p * (1.0 - ew_ref[...]) + jnp.log(1.0 + jnp.exp(-p))
        pos_loss = jnp.sum(jnp.where(valid, pos_t, 0.0)) / E
        qraw = jnp.dot(neg_ref[...], seg, preferred_element_type=jnp.float32)
        q = 1.0 / (1.0 + jnp.exp(-qraw))
        neg_t = jnp.log(1.0 + jnp.exp(q))
        neg_loss = jnp.sum(jnp.where(valid, neg_t, 0.0)) / E
        out_ref[...] = (pos_loss + neg_loss) + kl_ref[...] / N

    return pl.pallas_call(
        body,
        out_shape=jax.ShapeDtypeStruct((1, 1), jnp.float32),
    )(pos_pp, neg_pp, ew8, kl)


def kernel(x, edge_index, edge_weight, W_enc1, b_enc1, g_enc1, be_enc1,
           W_enc2, b_enc2, g_enc2, be_enc2, W_gf, b_gf, W_gm, b_gm,
           W_gv, b_gv, W_dec, b_dec, g_dec, be_dec):
    src = edge_index[0].astype(jnp.int32)
    dst = edge_index[1].astype(jnp.int32)
    ew = edge_weight.astype(jnp.float32)

    pad = EP - E
    zpad_i = jnp.zeros((pad,), jnp.int32)
    srcp = jnp.concatenate([src, zpad_i]).reshape(NT, C_CONV, CH)
    dstp = jnp.concatenate([dst, zpad_i]).reshape(NT, C_CONV, CH)
    ewp = jnp.concatenate([ew, jnp.zeros((pad,), jnp.float32)]
                          ).reshape(NT, C_CONV, CH)
    zero1 = jnp.zeros((N,), jnp.float32)
    zero32 = jnp.zeros((N, 32), jnp.float32)
    zero16 = jnp.zeros((N, 16), jnp.float32)

    degh = _sc_deg(ewp, dstp, zero1)
    featx, g1m, gs1, dinv = _tc_encoder(
        x, W_enc1, b_enc1, g_enc1, be_enc1,
        W_enc2, b_enc2, g_enc2, be_enc2, W_gf, degh.reshape(2, N, 1))
    acc1 = _sc_conv(gs1, srcp, dstp, ewp, zero32, 32)
    g2m, gs2 = _tc_mid(acc1, g1m, dinv, b_gf, W_gm, W_gv)
    acc2 = _sc_conv(gs2, srcp, dstp, ewp, zero16, 16)
    feat, dae, kl = _tc_final(
        acc2, g2m, dinv, b_gm, b_gv, featx, W_dec, b_dec, g_dec, be_dec, x)

    neg = jax.random.randint(jax.random.key(42), (2, E), 0, N
                             ).astype(jnp.int32)
    nega = jnp.concatenate([neg[0], zpad_i]).reshape(C_TOT, CH)
    negb = jnp.concatenate([neg[1], zpad_i]).reshape(C_TOT, CH)
    out_pos, out_neg = _sc_dots(
        feat, srcp.reshape(C_TOT, CH), dstp.reshape(C_TOT, CH), nega, negb)
    pos_pp = out_pos.reshape(EP // 8, 128)
    neg_pp = out_neg.reshape(EP // 8, 128)
    gae = _tc_loss(pos_pp, neg_pp, ewp.reshape(EP // 8, 8), kl)

    return (feat, dae.reshape(()), gae.reshape(()))


# speedup vs baseline: 1.0889x; 1.0099x over previous
import functools

import jax
import jax.numpy as jnp
from jax import lax
from jax.experimental import pallas as pl
from jax.experimental.pallas import tpu as pltpu
from jax.experimental.pallas import tpu_sc as plsc

N = 10000
E = 160000
NT = 32
CH = 128
C_CONV = 40
EP = NT * C_CONV * CH
C_DOT = 80
EP2 = NT * C_DOT * CH


def _sc_mesh():
    return plsc.VectorSubcoreMesh(core_axis_name="c", subcore_axis_name="s")


_SC_PARAMS = pltpu.CompilerParams(
    use_tc_tiling_on_sc=False, needs_layout_passes=False)


def _sc_deg(ew_rows, dst_idx, zero1):
    @functools.partial(
        pl.kernel,
        out_type=jax.ShapeDtypeStruct((2, N), jnp.float32),
        mesh=_sc_mesh(),
        compiler_params=_SC_PARAMS,
        scratch_types=[
            pltpu.VMEM((C_CONV, CH), jnp.float32),
            pltpu.VMEM((C_CONV, CH), jnp.int32),
            pltpu.VMEM_SHARED((N,), jnp.float32),
        ],
    )
    def k(ew_hbm, dst_hbm, zero_hbm, out_hbm, ew_v, dst_v, acc):
        c = lax.axis_index("c")
        s = lax.axis_index("s")
        t = s * 2 + c

        @pl.when(s == 0)
        def _():
            pltpu.sync_copy(zero_hbm, acc)

        plsc.subcore_barrier()
        pltpu.sync_copy(ew_hbm.at[t], ew_v)
        pltpu.sync_copy(dst_hbm.at[t], dst_v)

        def body(j, carry):
            pltpu.sync_copy(ew_v.at[j], acc.at[dst_v.at[j]], add=True)
            return carry

        lax.fori_loop(0, C_CONV, body, 0)
        plsc.subcore_barrier()

        @pl.when(s == 0)
        def _():
            pltpu.sync_copy(acc, out_hbm.at[c])

    return k(ew_rows, dst_idx, zero1)


def _sc_conv(tbl, src_idx, dst_idx, ew_rows, zero_w, width):
    @functools.partial(
        pl.kernel,
        out_type=jax.ShapeDtypeStruct((2, N, width), jnp.float32),
        mesh=_sc_mesh(),
        compiler_params=_SC_PARAMS,
        scratch_types=[
            pltpu.VMEM((C_CONV, CH), jnp.int32),
            pltpu.VMEM((C_CONV, CH), jnp.int32),
            pltpu.VMEM((C_CONV, CH), jnp.float32),
            pltpu.VMEM((CH, width), jnp.float32),
            pltpu.VMEM((CH, width), jnp.float32),
            pltpu.VMEM_SHARED((N, width), jnp.float32),
            pltpu.SemaphoreType.DMA,
            pltpu.SemaphoreType.DMA,
        ],
    )
    def k(tbl_hbm, src_hbm, dst_hbm, ew_hbm, zero_hbm, out_hbm,
          src_v, dst_v, ew_v, gb0, gb1, acc, sg0, sg1):
        c = lax.axis_index("c")
        s = lax.axis_index("s")
        t = s * 2 + c

        @pl.when(s == 0)
        def _():
            pltpu.sync_copy(zero_hbm, acc)

        plsc.subcore_barrier()
        pltpu.sync_copy(src_hbm.at[t], src_v)
        pltpu.sync_copy(dst_hbm.at[t], dst_v)
        pltpu.sync_copy(ew_hbm.at[t], ew_v)

        pltpu.async_copy(tbl_hbm.at[src_v.at[0]], gb0, sg0)
        pltpu.async_copy(tbl_hbm.at[src_v.at[1]], gb1, sg1)

        def half(jj, gbuf, sg):
            pltpu.make_async_copy(tbl_hbm.at[src_v.at[jj]], gbuf, sg).wait()

            def scale(eb, cc):
                wv = ew_v[jj, pl.ds(eb * 16, 16)]
                for lane in range(16):
                    e = eb * 16 + lane
                    w = wv[lane]
                    for h in range(width // 16):
                        gbuf[e, pl.ds(h * 16, 16)] = (
                            gbuf[e, pl.ds(h * 16, 16)] * w)
                return cc

            lax.fori_loop(0, CH // 16, scale, 0)
            pltpu.sync_copy(gbuf, acc.at[dst_v.at[jj]], add=True)

            @pl.when(jj + 2 < C_CONV)
            def _():
                pltpu.async_copy(tbl_hbm.at[src_v.at[jj + 2]], gbuf, sg)

        def body(j, carry):
            half(2 * j, gb0, sg0)
            half(2 * j + 1, gb1, sg1)
            return carry

        lax.fori_loop(0, C_CONV // 2, body, 0)
        plsc.subcore_barrier()

        @pl.when(s == 0)
        def _():
            pltpu.sync_copy(acc, out_hbm.at[c])

    return k(tbl, src_idx, dst_idx, ew_rows, zero_w)


C_TOT = NT * C_CONV
CS = 32
CF = C_CONV * 2 - CS
SLOW_CORE = 1


def _sc_dots(tbl, pa_idx, pb_idx, na_idx, nb_idx):
    @functools.partial(
        pl.kernel,
        out_type=[
            jax.ShapeDtypeStruct((C_TOT, CH, 16), jnp.float32),
            jax.ShapeDtypeStruct((C_TOT, CH, 16), jnp.float32),
        ],
        mesh=_sc_mesh(),
        compiler_params=_SC_PARAMS,
        scratch_types=[
            pltpu.VMEM((CF, CH), jnp.int32),
            pltpu.VMEM((CF, CH), jnp.int32),
            pltpu.VMEM((CF, CH), jnp.int32),
            pltpu.VMEM((CF, CH), jnp.int32),
            pltpu.VMEM((CH, 32), jnp.float32),
            pltpu.VMEM((CH, 32), jnp.float32),
            pltpu.VMEM((CH, 32), jnp.float32),
            pltpu.VMEM((CH, 32), jnp.float32),
            pltpu.VMEM((CH, 16), jnp.float32),
            pltpu.VMEM((CH, 16), jnp.float32),
            pltpu.SemaphoreType.DMA,
            pltpu.SemaphoreType.DMA,
            pltpu.SemaphoreType.DMA,
            pltpu.SemaphoreType.DMA,
        ],
    )
    def k(tbl_hbm, pa_hbm, pb_hbm, na_hbm, nb_hbm, opos_hbm, oneg_hbm,
          pav, pbv, nav, nbv, abA, bbA, abB, bbB, pbA, pbB,
          sgA, sgB, swA, swB):
        c = lax.axis_index("c")
        s = lax.axis_index("s")

        def run(lo, cnt):
            pltpu.sync_copy(pa_hbm.at[pl.ds(lo, cnt)], pav.at[pl.ds(0, cnt)])
            pltpu.sync_copy(pb_hbm.at[pl.ds(lo, cnt)], pbv.at[pl.ds(0, cnt)])
            pltpu.sync_copy(na_hbm.at[pl.ds(lo, cnt)], nav.at[pl.ds(0, cnt)])
            pltpu.sync_copy(nb_hbm.at[pl.ds(lo, cnt)], nbv.at[pl.ds(0, cnt)])

            pltpu.async_copy(tbl_hbm.at[pav.at[0]], abA, sgA)
            pltpu.async_copy(tbl_hbm.at[pbv.at[0]], bbA, sgA)
            pltpu.async_copy(tbl_hbm.at[nav.at[0]], abB, sgB)
            pltpu.async_copy(tbl_hbm.at[nbv.at[0]], bbB, sgB)

            def half(j, av, bv, abuf, bbuf, pbuf, sg, sw, out_hbm):
                pltpu.make_async_copy(tbl_hbm.at[av.at[j]], abuf, sg).wait()
                pltpu.make_async_copy(tbl_hbm.at[bv.at[j]], bbuf, sg).wait()

                @pl.when(j > 0)
                def _():
                    pltpu.make_async_copy(
                        pbuf, out_hbm.at[lo + j], sw).wait()

                def edge(e, cc):
                    pbuf[e, pl.ds(0, 16)] = (
                        abuf[e, pl.ds(0, 16)] * bbuf[e, pl.ds(0, 16)]
                        + abuf[e, pl.ds(16, 16)] * bbuf[e, pl.ds(16, 16)])
                    return cc

                lax.fori_loop(0, CH, edge, 0, unroll=16)
                pltpu.async_copy(pbuf, out_hbm.at[lo + j], sw)

                @pl.when(j + 1 < cnt)
                def _():
                    pltpu.async_copy(tbl_hbm.at[av.at[j + 1]], abuf, sg)
                    pltpu.async_copy(tbl_hbm.at[bv.at[j + 1]], bbuf, sg)

            def body(j, carry):
                half(j, pav, pbv, abA, bbA, pbA, sgA, swA, opos_hbm)
                half(j, nav, nbv, abB, bbB, pbB, sgB, swB, oneg_hbm)
                return carry

            lax.fori_loop(0, cnt, body, 0)
            pltpu.make_async_copy(pbA, opos_hbm.at[lo + cnt - 1], swA).wait()
            pltpu.make_async_copy(pbB, oneg_hbm.at[lo + cnt - 1], swB).wait()

        @pl.when(c == SLOW_CORE)
        def _():
            run(s * CS, CS)

        @pl.when(c != SLOW_CORE)
        def _():
            run(16 * CS + s * CF, CF)

    return k(tbl, pa_idx, pb_idx, na_idx, nb_idx)


def _bn_elu(h, g, b):
    m = jnp.mean(h, axis=0)
    v = jnp.mean((h - m) ** 2, axis=0)
    hn = (h - m) * lax.rsqrt(v + 1e-3) * g + b
    return jnp.where(hn > 0, hn, jnp.exp(hn) - 1.0)


def _tc_encoder(x, W1, b1, g1, be1, W2, b2, g2, be2, Wgf, degh):
    def body(x_ref, W1_ref, b1_ref, g1_ref, be1_ref, W2_ref, b2_ref,
             g2_ref, be2_ref, Wgf_ref, degh_ref,
             featx_ref, g1o_ref, gs1_ref, dinv_ref):
        h1 = _bn_elu(
            jnp.dot(x_ref[...], W1_ref[...],
                    preferred_element_type=jnp.float32) + b1_ref[...],
            g1_ref[...], be1_ref[...])
        fx = _bn_elu(
            jnp.dot(h1, W2_ref[...],
                    preferred_element_type=jnp.float32) + b2_ref[...],
            g2_ref[...], be2_ref[...])
        featx_ref[...] = fx
        deg = degh_ref[0] + degh_ref[1] + 1.0
        dinv = lax.rsqrt(deg)
        gv = jnp.dot(fx, Wgf_ref[...], preferred_element_type=jnp.float32)
        g1o_ref[...] = gv
        gs1_ref[...] = gv * dinv
        dinv_ref[...] = dinv

    return pl.pallas_call(
        body,
        out_shape=[
            jax.ShapeDtypeStruct((N, 24), jnp.float32),
            jax.ShapeDtypeStruct((N, 32), jnp.float32),
            jax.ShapeDtypeStruct((N, 32), jnp.float32),
            jax.ShapeDtypeStruct((N, 1), jnp.float32),
        ],
    )(x, W1, b1, g1, be1, W2, b2, g2, be2, Wgf, degh)


def _tc_mid(acc1h, g1m, dinv, b_gf, Wgm, Wgv):
    def body(acc_ref, g1_ref, dinv_ref, bgf_ref, Wgm_ref, Wgv_ref,
             g2o_ref, gs2_ref):
        dinv = dinv_ref[...]
        a = acc_ref[0] + acc_ref[1]
        h = jnp.maximum(
            a * dinv + g1_ref[...] * (dinv * dinv) + bgf_ref[...], 0.0)
        g2m = jnp.dot(h, Wgm_ref[...], preferred_element_type=jnp.float32)
        g2v = jnp.dot(h, Wgv_ref[...], preferred_element_type=jnp.float32)
        g2 = jnp.concatenate([g2m, g2v], axis=1)
        g2o_ref[...] = g2
        gs2_ref[...] = g2 * dinv

    return pl.pallas_call(
        body,
        out_shape=[
            jax.ShapeDtypeStruct((N, 16), jnp.float32),
            jax.ShapeDtypeStruct((N, 16), jnp.float32),
        ],
    )(acc1h, g1m, dinv, b_gf, Wgm, Wgv)


def _tc_final(acc2h, g2m, dinv, b_gm, b_gv, featx, Wd, bd, gd, bed, x):
    def body(acc_ref, g2_ref, dinv_ref, bgm_ref, bgv_ref, featx_ref,
             Wd_ref, bd_ref, gd_ref, bed_ref, x_ref,
             feat_ref, dae_ref, kl_ref):
        dinv = dinv_ref[...]
        tt = (acc_ref[0] + acc_ref[1]) * dinv + g2_ref[...] * (dinv * dinv)
        mu = tt[:, :8] + bgm_ref[...]
        ls = jnp.minimum(tt[:, 8:] + bgv_ref[...], 10.0)
        feat = jnp.concatenate([featx_ref[...], mu], axis=1)
        feat_ref[...] = feat
        xd = _bn_elu(
            jnp.dot(feat, Wd_ref[...],
                    preferred_element_type=jnp.float32) + bd_ref[...],
            gd_ref[...], bed_ref[...])
        dae_ref[...] = jnp.reshape(jnp.mean((xd - x_ref[...]) ** 2), (1, 1))
        kl_ref[...] = jnp.reshape(-0.5 * jnp.mean(
            jnp.sum(1.0 + 2.0 * ls - mu * mu - jnp.exp(2.0 * ls), axis=1)),
            (1, 1))

    return pl.pallas_call(
        body,
        out_shape=[
            jax.ShapeDtypeStruct((N, 32), jnp.float32),
            jax.ShapeDtypeStruct((1, 1), jnp.float32),
            jax.ShapeDtypeStruct((1, 1), jnp.float32),
        ],
    )(acc2h, g2m, dinv, b_gm, b_gv, featx, Wd, bd, gd, bed, x)


def _tc_loss(pos_pp, neg_pp, ew8, kl):
    rows = EP // 8
    vrows = E // 8

    def body(pos_ref, neg_ref, ew_ref, kl_ref, out_ref):
        seg = (lax.broadcasted_iota(jnp.int32, (128, 8), 0) // 16
               == lax.broadcasted_iota(jnp.int32, (128, 8), 1)
               ).astype(jnp.float32)
        valid = lax.broadcasted_iota(jnp.int32, (rows, 8), 0) < vrows
        praw = jnp.dot(pos_ref[...], seg, preferred_element_type=jnp.float32)
        p = 1.0 / (1.0 + jnp.exp(-praw))
        pos_t = p * (1.0 - ew_ref[...]) + jnp.log(1.0 + jnp.exp(-p))
        pos_loss = jnp.sum(jnp.where(valid, pos_t, 0.0)) / E
        qraw = jnp.dot(neg_ref[...], seg, preferred_element_type=jnp.float32)
        q = 1.0 / (1.0 + jnp.exp(-qraw))
        neg_t = jnp.log(1.0 + jnp.exp(q))
        neg_loss = jnp.sum(jnp.where(valid, neg_t, 0.0)) / E
        out_ref[...] = (pos_loss + neg_loss) + kl_ref[...] / N

    return pl.pallas_call(
        body,
        out_shape=jax.ShapeDtypeStruct((1, 1), jnp.float32),
    )(pos_pp, neg_pp, ew8, kl)


def kernel(x, edge_index, edge_weight, W_enc1, b_enc1, g_enc1, be_enc1,
           W_enc2, b_enc2, g_enc2, be_enc2, W_gf, b_gf, W_gm, b_gm,
           W_gv, b_gv, W_dec, b_dec, g_dec, be_dec):
    src = edge_index[0].astype(jnp.int32)
    dst = edge_index[1].astype(jnp.int32)
    ew = edge_weight.astype(jnp.float32)

    pad = EP - E
    zpad_i = jnp.zeros((pad,), jnp.int32)
    srcp = jnp.concatenate([src, zpad_i]).reshape(NT, C_CONV, CH)
    dstp = jnp.concatenate([dst, zpad_i]).reshape(NT, C_CONV, CH)
    ewp = jnp.concatenate([ew, jnp.zeros((pad,), jnp.float32)]
                          ).reshape(NT, C_CONV, CH)
    zero1 = jnp.zeros((N,), jnp.float32)
    zero32 = jnp.zeros((N, 32), jnp.float32)
    zero16 = jnp.zeros((N, 16), jnp.float32)

    degh = _sc_deg(ewp, dstp, zero1)
    featx, g1m, gs1, dinv = _tc_encoder(
        x, W_enc1, b_enc1, g_enc1, be_enc1,
        W_enc2, b_enc2, g_enc2, be_enc2, W_gf, degh.reshape(2, N, 1))
    acc1 = _sc_conv(gs1, srcp, dstp, ewp, zero32, 32)
    g2m, gs2 = _tc_mid(acc1, g1m, dinv, b_gf, W_gm, W_gv)
    acc2 = _sc_conv(gs2, srcp, dstp, ewp, zero16, 16)
    feat, dae, kl = _tc_final(
        acc2, g2m, dinv, b_gm, b_gv, featx, W_dec, b_dec, g_dec, be_dec, x)

    neg = jax.random.randint(jax.random.key(42), (2, E), 0, N
                             ).astype(jnp.int32)
    nega = jnp.concatenate([neg[0], zpad_i]).reshape(C_TOT, CH)
    negb = jnp.concatenate([neg[1], zpad_i]).reshape(C_TOT, CH)
    out_pos, out_neg = _sc_dots(
        feat, srcp.reshape(C_TOT, CH), dstp.reshape(C_TOT, CH), nega, negb)
    pos_pp = out_pos.reshape(EP // 8, 128)
    neg_pp = out_neg.reshape(EP // 8, 128)
    gae = _tc_loss(pos_pp, neg_pp, ewp.reshape(EP // 8, 8), kl)

    return (feat, dae.reshape(()), gae.reshape(()))


# speedup vs baseline: 1.0929x; 1.0036x over previous
import functools

import jax
import jax.numpy as jnp
from jax import lax
from jax.experimental import pallas as pl
from jax.experimental.pallas import tpu as pltpu
from jax.experimental.pallas import tpu_sc as plsc

N = 10000
E = 160000
NT = 32
CH = 128
C_CONV = 40
EP = NT * C_CONV * CH
C_DOT = 80
EP2 = NT * C_DOT * CH


def _sc_mesh():
    return plsc.VectorSubcoreMesh(core_axis_name="c", subcore_axis_name="s")


_SC_PARAMS = pltpu.CompilerParams(
    use_tc_tiling_on_sc=False, needs_layout_passes=False)


def _sc_deg(ew_rows, dst_idx, zero1):
    @functools.partial(
        pl.kernel,
        out_type=jax.ShapeDtypeStruct((2, N), jnp.float32),
        mesh=_sc_mesh(),
        compiler_params=_SC_PARAMS,
        scratch_types=[
            pltpu.VMEM((C_CONV, CH), jnp.float32),
            pltpu.VMEM((C_CONV, CH), jnp.int32),
            pltpu.VMEM_SHARED((N,), jnp.float32),
        ],
    )
    def k(ew_hbm, dst_hbm, zero_hbm, out_hbm, ew_v, dst_v, acc):
        c = lax.axis_index("c")
        s = lax.axis_index("s")
        t = s * 2 + c

        @pl.when(s == 0)
        def _():
            pltpu.sync_copy(zero_hbm, acc)

        plsc.subcore_barrier()
        pltpu.sync_copy(ew_hbm.at[t], ew_v)
        pltpu.sync_copy(dst_hbm.at[t], dst_v)

        def body(j, carry):
            pltpu.sync_copy(ew_v.at[j], acc.at[dst_v.at[j]], add=True)
            return carry

        lax.fori_loop(0, C_CONV, body, 0)
        plsc.subcore_barrier()

        @pl.when(s == 0)
        def _():
            pltpu.sync_copy(acc, out_hbm.at[c])

    return k(ew_rows, dst_idx, zero1)


def _sc_conv(tbl, src_idx, dst_idx, ew_rows, zero_w, width):
    @functools.partial(
        pl.kernel,
        out_type=jax.ShapeDtypeStruct((2, N, width), jnp.float32),
        mesh=_sc_mesh(),
        compiler_params=_SC_PARAMS,
        scratch_types=[
            pltpu.VMEM((C_CONV, CH), jnp.int32),
            pltpu.VMEM((C_CONV, CH), jnp.int32),
            pltpu.VMEM((C_CONV, CH), jnp.float32),
            pltpu.VMEM((CH, width), jnp.float32),
            pltpu.VMEM((CH, width), jnp.float32),
            pltpu.VMEM_SHARED((N, width), jnp.float32),
            pltpu.SemaphoreType.DMA,
            pltpu.SemaphoreType.DMA,
        ],
    )
    def k(tbl_hbm, src_hbm, dst_hbm, ew_hbm, zero_hbm, out_hbm,
          src_v, dst_v, ew_v, gb0, gb1, acc, sg0, sg1):
        c = lax.axis_index("c")
        s = lax.axis_index("s")
        t = s * 2 + c

        @pl.when(s == 0)
        def _():
            pltpu.sync_copy(zero_hbm, acc)

        plsc.subcore_barrier()
        pltpu.sync_copy(src_hbm.at[t], src_v)
        pltpu.sync_copy(dst_hbm.at[t], dst_v)
        pltpu.sync_copy(ew_hbm.at[t], ew_v)

        pltpu.async_copy(tbl_hbm.at[src_v.at[0]], gb0, sg0)
        pltpu.async_copy(tbl_hbm.at[src_v.at[1]], gb1, sg1)

        def half(jj, gbuf, sg):
            pltpu.make_async_copy(tbl_hbm.at[src_v.at[jj]], gbuf, sg).wait()

            def scale(eb, cc):
                wv = ew_v[jj, pl.ds(eb * 16, 16)]
                for lane in range(16):
                    e = eb * 16 + lane
                    w = wv[lane]
                    for h in range(width // 16):
                        gbuf[e, pl.ds(h * 16, 16)] = (
                            gbuf[e, pl.ds(h * 16, 16)] * w)
                return cc

            lax.fori_loop(0, CH // 16, scale, 0)
            pltpu.sync_copy(gbuf, acc.at[dst_v.at[jj]], add=True)

            @pl.when(jj + 2 < C_CONV)
            def _():
                pltpu.async_copy(tbl_hbm.at[src_v.at[jj + 2]], gbuf, sg)

        def body(j, carry):
            half(2 * j, gb0, sg0)
            half(2 * j + 1, gb1, sg1)
            return carry

        lax.fori_loop(0, C_CONV // 2, body, 0)
        plsc.subcore_barrier()

        @pl.when(s == 0)
        def _():
            pltpu.sync_copy(acc, out_hbm.at[c])

    return k(tbl, src_idx, dst_idx, ew_rows, zero_w)


C_TOT = NT * C_CONV
CS = 40
CF = C_CONV * 2 - CS
SLOW_CORE = 1


def _sc_dots(tbl, pa_idx, pb_idx, na_idx, nb_idx):
    @functools.partial(
        pl.kernel,
        out_type=[
            jax.ShapeDtypeStruct((C_TOT, CH, 16), jnp.float32),
            jax.ShapeDtypeStruct((C_TOT, CH, 16), jnp.float32),
        ],
        mesh=_sc_mesh(),
        compiler_params=_SC_PARAMS,
        scratch_types=[
            pltpu.VMEM((CF, CH), jnp.int32),
            pltpu.VMEM((CF, CH), jnp.int32),
            pltpu.VMEM((CF, CH), jnp.int32),
            pltpu.VMEM((CF, CH), jnp.int32),
            pltpu.VMEM((CH, 32), jnp.float32),
            pltpu.VMEM((CH, 32), jnp.float32),
            pltpu.VMEM((CH, 32), jnp.float32),
            pltpu.VMEM((CH, 32), jnp.float32),
            pltpu.VMEM((CH, 16), jnp.float32),
            pltpu.VMEM((CH, 16), jnp.float32),
            pltpu.SemaphoreType.DMA,
            pltpu.SemaphoreType.DMA,
            pltpu.SemaphoreType.DMA,
            pltpu.SemaphoreType.DMA,
        ],
    )
    def k(tbl_hbm, pa_hbm, pb_hbm, na_hbm, nb_hbm, opos_hbm, oneg_hbm,
          pav, pbv, nav, nbv, abA, bbA, abB, bbB, pbA, pbB,
          sgA, sgB, swA, swB):
        c = lax.axis_index("c")
        s = lax.axis_index("s")

        def run(lo, cnt):
            pltpu.sync_copy(pa_hbm.at[pl.ds(lo, cnt)], pav.at[pl.ds(0, cnt)])
            pltpu.sync_copy(pb_hbm.at[pl.ds(lo, cnt)], pbv.at[pl.ds(0, cnt)])
            pltpu.sync_copy(na_hbm.at[pl.ds(lo, cnt)], nav.at[pl.ds(0, cnt)])
            pltpu.sync_copy(nb_hbm.at[pl.ds(lo, cnt)], nbv.at[pl.ds(0, cnt)])

            pltpu.async_copy(tbl_hbm.at[pav.at[0]], abA, sgA)
            pltpu.async_copy(tbl_hbm.at[pbv.at[0]], bbA, sgA)
            pltpu.async_copy(tbl_hbm.at[nav.at[0]], abB, sgB)
            pltpu.async_copy(tbl_hbm.at[nbv.at[0]], bbB, sgB)

            def half(j, av, bv, abuf, bbuf, pbuf, sg, sw, out_hbm):
                pltpu.make_async_copy(tbl_hbm.at[av.at[j]], abuf, sg).wait()
                pltpu.make_async_copy(tbl_hbm.at[bv.at[j]], bbuf, sg).wait()

                @pl.when(j > 0)
                def _():
                    pltpu.make_async_copy(
                        pbuf, out_hbm.at[lo + j], sw).wait()

                def edge(e, cc):
                    pbuf[e, pl.ds(0, 16)] = (
                        abuf[e, pl.ds(0, 16)] * bbuf[e, pl.ds(0, 16)]
                        + abuf[e, pl.ds(16, 16)] * bbuf[e, pl.ds(16, 16)])
                    return cc

                lax.fori_loop(0, CH, edge, 0, unroll=16)
                pltpu.async_copy(pbuf, out_hbm.at[lo + j], sw)

                @pl.when(j + 1 < cnt)
                def _():
                    pltpu.async_copy(tbl_hbm.at[av.at[j + 1]], abuf, sg)
                    pltpu.async_copy(tbl_hbm.at[bv.at[j + 1]], bbuf, sg)

            def body(j, carry):
                half(j, pav, pbv, abA, bbA, pbA, sgA, swA, opos_hbm)
                half(j, nav, nbv, abB, bbB, pbB, sgB, swB, oneg_hbm)
                return carry

            lax.fori_loop(0, cnt, body, 0)
            pltpu.make_async_copy(pbA, opos_hbm.at[lo + cnt - 1], swA).wait()
            pltpu.make_async_copy(pbB, oneg_hbm.at[lo + cnt - 1], swB).wait()

        @pl.when(c == SLOW_CORE)
        def _():
            run(s * CS, CS)

        @pl.when(c != SLOW_CORE)
        def _():
            run(16 * CS + s * CF, CF)

    return k(tbl, pa_idx, pb_idx, na_idx, nb_idx)


def _bn_elu(h, g, b):
    m = jnp.mean(h, axis=0)
    v = jnp.mean((h - m) ** 2, axis=0)
    hn = (h - m) * lax.rsqrt(v + 1e-3) * g + b
    return jnp.where(hn > 0, hn, jnp.exp(hn) - 1.0)


def _tc_encoder(x, W1, b1, g1, be1, W2, b2, g2, be2, Wgf, degh):
    def body(x_ref, W1_ref, b1_ref, g1_ref, be1_ref, W2_ref, b2_ref,
             g2_ref, be2_ref, Wgf_ref, degh_ref,
             featx_ref, g1o_ref, gs1_ref, dinv_ref):
        h1 = _bn_elu(
            jnp.dot(x_ref[...], W1_ref[...],
                    preferred_element_type=jnp.float32) + b1_ref[...],
            g1_ref[...], be1_ref[...])
        fx = _bn_elu(
            jnp.dot(h1, W2_ref[...],
                    preferred_element_type=jnp.float32) + b2_ref[...],
            g2_ref[...], be2_ref[...])
        featx_ref[...] = fx
        deg = degh_ref[0] + degh_ref[1] + 1.0
        dinv = lax.rsqrt(deg)
        gv = jnp.dot(fx, Wgf_ref[...], preferred_element_type=jnp.float32)
        g1o_ref[...] = gv
        gs1_ref[...] = gv * dinv
        dinv_ref[...] = dinv

    return pl.pallas_call(
        body,
        out_shape=[
            jax.ShapeDtypeStruct((N, 24), jnp.float32),
            jax.ShapeDtypeStruct((N, 32), jnp.float32),
            jax.ShapeDtypeStruct((N, 32), jnp.float32),
            jax.ShapeDtypeStruct((N, 1), jnp.float32),
        ],
    )(x, W1, b1, g1, be1, W2, b2, g2, be2, Wgf, degh)


def _tc_mid(acc1h, g1m, dinv, b_gf, Wgm, Wgv):
    def body(acc_ref, g1_ref, dinv_ref, bgf_ref, Wgm_ref, Wgv_ref,
             g2o_ref, gs2_ref):
        dinv = dinv_ref[...]
        a = acc_ref[0] + acc_ref[1]
        h = jnp.maximum(
            a * dinv + g1_ref[...] * (dinv * dinv) + bgf_ref[...], 0.0)
        g2m = jnp.dot(h, Wgm_ref[...], preferred_element_type=jnp.float32)
        g2v = jnp.dot(h, Wgv_ref[...], preferred_element_type=jnp.float32)
        g2 = jnp.concatenate([g2m, g2v], axis=1)
        g2o_ref[...] = g2
        gs2_ref[...] = g2 * dinv

    return pl.pallas_call(
        body,
        out_shape=[
            jax.ShapeDtypeStruct((N, 16), jnp.float32),
            jax.ShapeDtypeStruct((N, 16), jnp.float32),
        ],
    )(acc1h, g1m, dinv, b_gf, Wgm, Wgv)


def _tc_final(acc2h, g2m, dinv, b_gm, b_gv, featx, Wd, bd, gd, bed, x):
    def body(acc_ref, g2_ref, dinv_ref, bgm_ref, bgv_ref, featx_ref,
             Wd_ref, bd_ref, gd_ref, bed_ref, x_ref,
             feat_ref, dae_ref, kl_ref):
        dinv = dinv_ref[...]
        tt = (acc_ref[0] + acc_ref[1]) * dinv + g2_ref[...] * (dinv * dinv)
        mu = tt[:, :8] + bgm_ref[...]
        ls = jnp.minimum(tt[:, 8:] + bgv_ref[...], 10.0)
        feat = jnp.concatenate([featx_ref[...], mu], axis=1)
        feat_ref[...] = feat
        xd = _bn_elu(
            jnp.dot(feat, Wd_ref[...],
                    preferred_element_type=jnp.float32) + bd_ref[...],
            gd_ref[...], bed_ref[...])
        dae_ref[...] = jnp.reshape(jnp.mean((xd - x_ref[...]) ** 2), (1, 1))
        kl_ref[...] = jnp.reshape(-0.5 * jnp.mean(
            jnp.sum(1.0 + 2.0 * ls - mu * mu - jnp.exp(2.0 * ls), axis=1)),
            (1, 1))

    return pl.pallas_call(
        body,
        out_shape=[
            jax.ShapeDtypeStruct((N, 32), jnp.float32),
            jax.ShapeDtypeStruct((1, 1), jnp.float32),
            jax.ShapeDtypeStruct((1, 1), jnp.float32),
        ],
    )(acc2h, g2m, dinv, b_gm, b_gv, featx, Wd, bd, gd, bed, x)


def _tc_loss(pos_pp, neg_pp, ew8, kl):
    rows = EP // 8
    vrows = E // 8

    def body(pos_ref, neg_ref, ew_ref, kl_ref, out_ref):
        seg = (lax.broadcasted_iota(jnp.int32, (128, 8), 0) // 16
               == lax.broadcasted_iota(jnp.int32, (128, 8), 1)
               ).astype(jnp.float32)
        valid = lax.broadcasted_iota(jnp.int32, (rows, 8), 0) < vrows
        praw = jnp.dot(pos_ref[...], seg, preferred_element_type=jnp.float32)
        p = 1.0 / (1.0 + jnp.exp(-praw))
        pos_t = p * (1.0 - ew_ref[...]) + jnp.log(1.0 + jnp.exp(-p))
        pos_loss = jnp.sum(jnp.where(valid, pos_t, 0.0)) / E
        qraw = jnp.dot(neg_ref[...], seg, preferred_element_type=jnp.float32)
        q = 1.0 / (1.0 + jnp.exp(-qraw))
        neg_t = jnp.log(1.0 + jnp.exp(q))
        neg_loss = jnp.sum(jnp.where(valid, neg_t, 0.0)) / E
        out_ref[...] = (pos_loss + neg_loss) + kl_ref[...] / N

    return pl.pallas_call(
        body,
        out_shape=jax.ShapeDtypeStruct((1, 1), jnp.float32),
    )(pos_pp, neg_pp, ew8, kl)


def kernel(x, edge_index, edge_weight, W_enc1, b_enc1, g_enc1, be_enc1,
           W_enc2, b_enc2, g_enc2, be_enc2, W_gf, b_gf, W_gm, b_gm,
           W_gv, b_gv, W_dec, b_dec, g_dec, be_dec):
    src = edge_index[0].astype(jnp.int32)
    dst = edge_index[1].astype(jnp.int32)
    ew = edge_weight.astype(jnp.float32)

    pad = EP - E
    zpad_i = jnp.zeros((pad,), jnp.int32)
    srcp = jnp.concatenate([src, zpad_i]).reshape(NT, C_CONV, CH)
    dstp = jnp.concatenate([dst, zpad_i]).reshape(NT, C_CONV, CH)
    ewp = jnp.concatenate([ew, jnp.zeros((pad,), jnp.float32)]
                          ).reshape(NT, C_CONV, CH)
    zero1 = jnp.zeros((N,), jnp.float32)
    zero32 = jnp.zeros((N, 32), jnp.float32)
    zero16 = jnp.zeros((N, 16), jnp.float32)

    degh = _sc_deg(ewp, dstp, zero1)
    featx, g1m, gs1, dinv = _tc_encoder(
        x, W_enc1, b_enc1, g_enc1, be_enc1,
        W_enc2, b_enc2, g_enc2, be_enc2, W_gf, degh.reshape(2, N, 1))
    acc1 = _sc_conv(gs1, srcp, dstp, ewp, zero32, 32)
    g2m, gs2 = _tc_mid(acc1, g1m, dinv, b_gf, W_gm, W_gv)
    acc2 = _sc_conv(gs2, srcp, dstp, ewp, zero16, 16)
    feat, dae, kl = _tc_final(
        acc2, g2m, dinv, b_gm, b_gv, featx, W_dec, b_dec, g_dec, be_dec, x)

    neg = jax.random.randint(jax.random.key(42), (2, E), 0, N
                             ).astype(jnp.int32)
    nega = jnp.concatenate([neg[0], zpad_i]).reshape(C_TOT, CH)
    negb = jnp.concatenate([neg[1], zpad_i]).reshape(C_TOT, CH)
    out_pos, out_neg = _sc_dots(
        feat, srcp.reshape(C_TOT, CH), dstp.reshape(C_TOT, CH), nega, negb)
    pos_pp = out_pos.reshape(EP // 8, 128)
    neg_pp = out_neg.reshape(EP // 8, 128)
    gae = _tc_loss(pos_pp, neg_pp, ewp.reshape(EP // 8, 8), kl)

    return (feat, dae.reshape(()), gae.reshape(()))
